# baseline (speedup 1.0000x reference)
import functools

import jax
import jax.numpy as jnp
from jax import lax
from jax.experimental import pallas as pl
from jax.experimental.pallas import tpu as pltpu
from jax.experimental.pallas import tpu_sc as plsc

_NUM_FIELDS = 26
_VOCAB = 100000
_EMBED = 32
_BATCH = 16384

_INFO = plsc.get_sparse_core_info()
_NC = _INFO.num_cores
_NS = _INFO.num_subcores
_NW = _NC * _NS
_TOTAL = _NUM_FIELDS * _BATCH
_PER_W = _TOTAL // _NW
_CHUNK = 1664
_NCHUNK = _PER_W // _CHUNK


@functools.partial(
    pl.kernel,
    mesh=plsc.VectorSubcoreMesh(core_axis_name="c", subcore_axis_name="s"),
    out_type=jax.ShapeDtypeStruct((_TOTAL, _EMBED), jnp.float32),
    scratch_types=[
        pltpu.VMEM((_CHUNK,), jnp.int32),
        pltpu.VMEM((_CHUNK, _EMBED), jnp.float32),
        pltpu.SemaphoreType.DMA,
    ],
    compiler_params=pltpu.CompilerParams(use_tc_tiling_on_sc=False),
)
def _gather_all(table_hbm, gidx_hbm, out_hbm, idx_v, rows_v, sem):
    wid = lax.axis_index("s") * _NC + lax.axis_index("c")
    base = wid * _PER_W
    for c in range(_NCHUNK):
        off = base + c * _CHUNK
        pltpu.sync_copy(gidx_hbm.at[pl.ds(off, _CHUNK)], idx_v)
        pltpu.async_copy(table_hbm.at[idx_v], rows_v, sem).wait()
        pltpu.sync_copy(rows_v, out_hbm.at[pl.ds(off, _CHUNK)])


def kernel(tables, x):
    flat = tables.reshape(_NUM_FIELDS * _VOCAB, _EMBED)
    offs = (jnp.arange(_NUM_FIELDS, dtype=jnp.int32) * _VOCAB)[:, None]
    gidx = (x.T.astype(jnp.int32) + offs).reshape(-1)
    out = _gather_all(flat, gidx)
    out = out.reshape(_NUM_FIELDS, _BATCH, _EMBED)
    return tuple(out[i] for i in range(_NUM_FIELDS))


# speedup vs baseline: 1.1804x; 1.1804x over previous
import functools

import jax
import jax.numpy as jnp
from jax import lax
from jax.experimental import pallas as pl
from jax.experimental.pallas import tpu as pltpu
from jax.experimental.pallas import tpu_sc as plsc

_NUM_FIELDS = 26
_VOCAB = 100000
_EMBED = 32
_BATCH = 16384

_INFO = plsc.get_sparse_core_info()
_NC = _INFO.num_cores
_NS = _INFO.num_subcores
_NW = _NC * _NS
_BPW = _BATCH // _NW
_L = 16


@functools.partial(
    pl.kernel,
    mesh=plsc.VectorSubcoreMesh(core_axis_name="c", subcore_axis_name="s"),
    out_type=tuple(
        jax.ShapeDtypeStruct((_BATCH, _EMBED), jnp.float32)
        for _ in range(_NUM_FIELDS)
    ),
    scratch_types=[
        pltpu.VMEM((_BPW, _NUM_FIELDS), jnp.int32),
        pltpu.VMEM((_BPW,), jnp.int32),
        pltpu.VMEM((_BPW, _EMBED), jnp.float32),
        pltpu.SemaphoreType.DMA,
    ],
    compiler_params=pltpu.CompilerParams(
        use_tc_tiling_on_sc=False, needs_layout_passes=False
    ),
)
def _embed_all(table_hbm, x_hbm, *refs):
    outs = refs[:_NUM_FIELDS]
    xblk, idx_v, rows_v, sem = refs[_NUM_FIELDS:]
    wid = lax.axis_index("s") * _NC + lax.axis_index("c")
    base = wid * _BPW
    pltpu.sync_copy(x_hbm.at[pl.ds(base, _BPW)], xblk)
    lanes = lax.iota(jnp.int32, _L)
    for i in range(_NUM_FIELDS):
        col = jnp.full((_L,), i, jnp.int32)
        off = jnp.full((_L,), i * _VOCAB, jnp.int32)

        def build(j, _, col=col, off=off):
            rows = lanes + j * _L
            vals = plsc.load_gather(xblk, [rows, col])
            idx_v[pl.ds(j * _L, _L)] = vals + off
            return 0

        lax.fori_loop(0, _BPW // _L, build, 0)
        pltpu.async_copy(table_hbm.at[idx_v], rows_v, sem).wait()
        pltpu.sync_copy(rows_v, outs[i].at[pl.ds(base, _BPW)])


def kernel(tables, x):
    flat = tables.reshape(_NUM_FIELDS * _VOCAB, _EMBED)
    return _embed_all(flat, x)


# speedup vs baseline: 1.6618x; 1.4078x over previous
import functools

import jax
import jax.numpy as jnp
from jax import lax
from jax.experimental import pallas as pl
from jax.experimental.pallas import tpu as pltpu
from jax.experimental.pallas import tpu_sc as plsc

_NUM_FIELDS = 26
_VOCAB = 100000
_EMBED = 32
_BATCH = 16384

_INFO = plsc.get_sparse_core_info()
_NC = _INFO.num_cores
_NS = _INFO.num_subcores
_NW = _NC * _NS


@functools.partial(
    pl.kernel,
    mesh=plsc.VectorSubcoreMesh(core_axis_name="c", subcore_axis_name="s"),
    out_type=tuple(
        jax.ShapeDtypeStruct((_EMBED, _BATCH), jnp.float32)
        for _ in range(_NUM_FIELDS)
    ),
    scratch_types=[
        pltpu.VMEM((2, _BATCH), jnp.int32),
        pltpu.VMEM((2, _BATCH), jnp.float32),
        pltpu.SemaphoreType.DMA,
        pltpu.SemaphoreType.DMA,
        pltpu.SemaphoreType.DMA,
    ],
    compiler_params=pltpu.CompilerParams(
        use_tc_tiling_on_sc=False, needs_layout_passes=False
    ),
)
def _embed_all(table_t_hbm, x_t_hbm, *refs):
    outs = refs[:_NUM_FIELDS]
    idx_v, val_v, isem, gsem, osem = refs[_NUM_FIELDS:]
    e = lax.axis_index("s") * _NC + lax.axis_index("c")

    pltpu.async_copy(x_t_hbm.at[0], idx_v.at[0], isem).wait()
    pltpu.async_copy(x_t_hbm.at[1], idx_v.at[1], isem)
    pltpu.async_copy(table_t_hbm.at[e].at[idx_v.at[0]], val_v.at[0], gsem)
    for f in range(_NUM_FIELDS):
        b = f % 2
        nb = (f + 1) % 2
        pltpu.make_async_copy(table_t_hbm.at[0].at[idx_v.at[b]],
                              val_v.at[b], gsem).wait()
        if f + 1 < _NUM_FIELDS:
            pltpu.make_async_copy(x_t_hbm.at[0], idx_v.at[0], isem).wait()
            if f >= 1:
                pltpu.make_async_copy(val_v.at[0], outs[0].at[e], osem).wait()
            pltpu.async_copy(
                table_t_hbm.at[(f + 1) * _EMBED + e].at[idx_v.at[nb]],
                val_v.at[nb], gsem)
            if f + 2 < _NUM_FIELDS:
                pltpu.async_copy(x_t_hbm.at[f + 2], idx_v.at[b], isem)
        pltpu.async_copy(val_v.at[b], outs[f].at[e], osem)
    pltpu.make_async_copy(val_v.at[0], outs[0].at[e], osem).wait()
    pltpu.make_async_copy(val_v.at[0], outs[0].at[e], osem).wait()


def kernel(tables, x):
    table_t = tables.transpose(0, 2, 1).reshape(_NUM_FIELDS * _EMBED, _VOCAB)
    x_t = x.T
    outs_t = _embed_all(table_t, x_t)
    return tuple(o.T for o in outs_t)
